# initial kernel scaffold (unmeasured)
import jax
import jax.numpy as jnp
from jax import lax
from jax.experimental import pallas as pl
from jax.experimental.pallas import tpu as pltpu


def kernel(
    x,
):
    def body(*refs):
        pass

    out_shape = jax.ShapeDtypeStruct(..., jnp.float32)
    return pl.pallas_call(body, out_shape=out_shape)(...)


# baseline (device time: 117618 ns/iter reference)
import jax
import jax.numpy as jnp
from jax import lax
from jax.experimental import pallas as pl
from jax.experimental.pallas import tpu as pltpu


def kernel(x):
    x = x.astype(jnp.bfloat16)
    m, n = x.shape
    half = n // 2

    def body(x_ref, out_ref, send_sem, recv_sem, copy_sem):
        my_x = lax.axis_index("x")
        my_y = lax.axis_index("y")
        my_z = lax.axis_index("z")
        other_y = 1 - my_y
        partner = (my_x, other_y, my_z)

        barrier_sem = pltpu.get_barrier_semaphore()
        pl.semaphore_signal(
            barrier_sem, inc=1,
            device_id=partner, device_id_type=pl.DeviceIdType.MESH,
        )
        pl.semaphore_wait(barrier_sem, 1)

        local = pltpu.make_async_copy(
            x_ref.at[:, pl.ds(my_y * half, half)],
            out_ref.at[pl.ds(my_y * m, m), :],
            copy_sem,
        )
        local.start()

        rdma = pltpu.make_async_remote_copy(
            src_ref=x_ref.at[:, pl.ds(other_y * half, half)],
            dst_ref=out_ref.at[pl.ds(my_y * m, m), :],
            send_sem=send_sem,
            recv_sem=recv_sem,
            device_id=partner,
            device_id_type=pl.DeviceIdType.MESH,
        )
        rdma.start()

        local.wait()
        rdma.wait()

    return pl.pallas_call(
        body,
        out_shape=jax.ShapeDtypeStruct((2 * m, half), x.dtype),
        in_specs=[pl.BlockSpec(memory_space=pl.ANY)],
        out_specs=pl.BlockSpec(memory_space=pl.ANY),
        scratch_shapes=[
            pltpu.SemaphoreType.DMA,
            pltpu.SemaphoreType.DMA,
            pltpu.SemaphoreType.DMA,
        ],
        compiler_params=pltpu.CompilerParams(collective_id=0),
    )(x)


# device time: 101555 ns/iter; 1.1582x vs baseline; 1.1582x over previous
import jax
import jax.numpy as jnp
from jax import lax
from jax.experimental import pallas as pl
from jax.experimental.pallas import tpu as pltpu

CHUNKS = 8


def kernel(x):
    m, n = x.shape
    half = n // 2
    cr = m // CHUNKS

    def body(x_ref, out_ref, f32_buf, keep_buf, send_buf,
             in_sems, send_sems, local_sems, recv_sems):
        my_x = lax.axis_index("x")
        my_y = lax.axis_index("y")
        my_z = lax.axis_index("z")
        other_y = 1 - my_y
        partner = (my_x, other_y, my_z)

        def make_load(k):
            return pltpu.make_async_copy(
                x_ref.at[pl.ds(k * cr, cr), :],
                f32_buf.at[k % 2],
                in_sems.at[k % 2],
            )

        def make_local(k):
            return pltpu.make_async_copy(
                keep_buf.at[k % 2],
                out_ref.at[pl.ds(my_y * m + k * cr, cr), :],
                local_sems.at[k % 2],
            )

        def make_rdma(k):
            return pltpu.make_async_remote_copy(
                src_ref=send_buf.at[k % 2],
                dst_ref=out_ref.at[pl.ds(my_y * m + k * cr, cr), :],
                send_sem=send_sems.at[k % 2],
                recv_sem=recv_sems.at[k],
                device_id=partner,
                device_id_type=pl.DeviceIdType.MESH,
            )

        loads = [make_load(k) for k in range(CHUNKS)]
        locals_ = [make_local(k) for k in range(CHUNKS)]
        rdmas = [make_rdma(k) for k in range(CHUNKS)]

        loads[0].start()
        if CHUNKS > 1:
            loads[1].start()

        barrier_sem = pltpu.get_barrier_semaphore()
        pl.semaphore_signal(
            barrier_sem, inc=1,
            device_id=partner, device_id_type=pl.DeviceIdType.MESH,
        )
        pl.semaphore_wait(barrier_sem, 1)

        for k in range(CHUNKS):
            s = k % 2
            loads[k].wait()
            if k >= 2:
                rdmas[k - 2].wait_send()
                locals_[k - 2].wait()
            @pl.when(my_y == 0)
            def _():
                keep_buf[s] = f32_buf[s][:, :half].astype(jnp.bfloat16)
                send_buf[s] = f32_buf[s][:, half:].astype(jnp.bfloat16)

            @pl.when(my_y == 1)
            def _():
                keep_buf[s] = f32_buf[s][:, half:].astype(jnp.bfloat16)
                send_buf[s] = f32_buf[s][:, :half].astype(jnp.bfloat16)
            if k + 2 < CHUNKS:
                loads[k + 2].start()
            rdmas[k].start()
            locals_[k].start()

        for k in range(max(CHUNKS - 2, 0), CHUNKS):
            rdmas[k].wait_send()
            locals_[k].wait()
        for k in range(CHUNKS):
            rdmas[k].wait_recv()

    return pl.pallas_call(
        body,
        out_shape=jax.ShapeDtypeStruct((2 * m, half), jnp.bfloat16),
        in_specs=[pl.BlockSpec(memory_space=pl.ANY)],
        out_specs=pl.BlockSpec(memory_space=pl.ANY),
        scratch_shapes=[
            pltpu.VMEM((2, cr, n), jnp.float32),
            pltpu.VMEM((2, cr, half), jnp.bfloat16),
            pltpu.VMEM((2, cr, half), jnp.bfloat16),
            pltpu.SemaphoreType.DMA((2,)),
            pltpu.SemaphoreType.DMA((2,)),
            pltpu.SemaphoreType.DMA((2,)),
            pltpu.SemaphoreType.DMA((CHUNKS,)),
        ],
        compiler_params=pltpu.CompilerParams(collective_id=0),
    )(x)


# device time: 100863 ns/iter; 1.1661x vs baseline; 1.0069x over previous
import jax
import jax.numpy as jnp
from jax import lax
from jax.experimental import pallas as pl
from jax.experimental.pallas import tpu as pltpu

CHUNKS = 16


def kernel(x):
    m, n = x.shape
    half = n // 2
    cr = m // CHUNKS

    def body(x_ref, out_ref, f32_buf, keep_buf, send_buf,
             in_sems, send_sems, local_sems, recv_sems):
        my_x = lax.axis_index("x")
        my_y = lax.axis_index("y")
        my_z = lax.axis_index("z")
        other_y = 1 - my_y
        partner = (my_x, other_y, my_z)

        def make_load(k):
            return pltpu.make_async_copy(
                x_ref.at[pl.ds(k * cr, cr), :],
                f32_buf.at[k % 2],
                in_sems.at[k % 2],
            )

        def make_local(k):
            return pltpu.make_async_copy(
                keep_buf.at[k % 2],
                out_ref.at[pl.ds(my_y * m + k * cr, cr), :],
                local_sems.at[k % 2],
            )

        def make_rdma(k):
            return pltpu.make_async_remote_copy(
                src_ref=send_buf.at[k % 2],
                dst_ref=out_ref.at[pl.ds(my_y * m + k * cr, cr), :],
                send_sem=send_sems.at[k % 2],
                recv_sem=recv_sems.at[k],
                device_id=partner,
                device_id_type=pl.DeviceIdType.MESH,
            )

        loads = [make_load(k) for k in range(CHUNKS)]
        locals_ = [make_local(k) for k in range(CHUNKS)]
        rdmas = [make_rdma(k) for k in range(CHUNKS)]

        loads[0].start()
        if CHUNKS > 1:
            loads[1].start()

        barrier_sem = pltpu.get_barrier_semaphore()
        pl.semaphore_signal(
            barrier_sem, inc=1,
            device_id=partner, device_id_type=pl.DeviceIdType.MESH,
        )
        pl.semaphore_wait(barrier_sem, 1)

        for k in range(CHUNKS):
            s = k % 2
            loads[k].wait()
            if k >= 2:
                rdmas[k - 2].wait_send()
                locals_[k - 2].wait()
            @pl.when(my_y == 0)
            def _():
                keep_buf[s] = f32_buf[s][:, :half].astype(jnp.bfloat16)
                send_buf[s] = f32_buf[s][:, half:].astype(jnp.bfloat16)

            @pl.when(my_y == 1)
            def _():
                keep_buf[s] = f32_buf[s][:, half:].astype(jnp.bfloat16)
                send_buf[s] = f32_buf[s][:, :half].astype(jnp.bfloat16)
            if k + 2 < CHUNKS:
                loads[k + 2].start()
            rdmas[k].start()
            locals_[k].start()

        for k in range(max(CHUNKS - 2, 0), CHUNKS):
            rdmas[k].wait_send()
            locals_[k].wait()
        for k in range(CHUNKS):
            rdmas[k].wait_recv()

    return pl.pallas_call(
        body,
        out_shape=jax.ShapeDtypeStruct((2 * m, half), jnp.bfloat16),
        in_specs=[pl.BlockSpec(memory_space=pl.ANY)],
        out_specs=pl.BlockSpec(memory_space=pl.ANY),
        scratch_shapes=[
            pltpu.VMEM((2, cr, n), jnp.float32),
            pltpu.VMEM((2, cr, half), jnp.bfloat16),
            pltpu.VMEM((2, cr, half), jnp.bfloat16),
            pltpu.SemaphoreType.DMA((2,)),
            pltpu.SemaphoreType.DMA((2,)),
            pltpu.SemaphoreType.DMA((2,)),
            pltpu.SemaphoreType.DMA((CHUNKS,)),
        ],
        compiler_params=pltpu.CompilerParams(collective_id=0),
    )(x)
